# baseline (speedup 1.0000x reference)
import functools

import jax
import jax.numpy as jnp
from jax import lax
from jax.experimental import pallas as pl
from jax.experimental.pallas import tpu as pltpu
from jax.experimental.pallas import tpu_sc as plsc

NUM_EXPERTS = 8
TOTAL_EXPERTS = 8
HIDDEN = 2048
BEH_DIM = 64
INTER = 768
T = 2048

BM = 128
NBLK = T // BM
NSTEPS = NBLK + TOTAL_EXPERTS - 1

BEH_PAD = 128

NW = 32
ROWS_PER_W = T // NW
CH = 16
NCH = ROWS_PER_W // CH


def _route_meta(action_index, position_index):
    idx = jnp.maximum(
        (NUM_EXPERTS - 1) * (action_index.astype(jnp.int32) - 1)
        + position_index.astype(jnp.int32), 0)
    eids = jnp.arange(TOTAL_EXPERTS, dtype=jnp.int32)
    oh = (idx[None, :] == eids[:, None]).astype(jnp.int32)
    csum = jnp.cumsum(oh, axis=1)
    counts = csum[:, -1]
    ends = jnp.cumsum(counts)
    starts = ends - counts
    cnt_before = jnp.take_along_axis(csum, idx[None, :], axis=0)[0] - 1
    rank = starts[idx] + cnt_before
    bfirst = starts // BM
    bcnt = jnp.where(counts > 0, (ends + BM - 1) // BM - bfirst, 0)
    co = jnp.cumsum(bcnt)
    s_ids = jnp.arange(NSTEPS, dtype=jnp.int32)
    e_s = jnp.sum((s_ids[:, None] >= co[None, :]).astype(jnp.int32), axis=1)
    total = co[TOTAL_EXPERTS - 1]
    valid = s_ids < total
    e_c = jnp.minimum(e_s, TOTAL_EXPERTS - 1)
    prev = jnp.where(e_c > 0, co[jnp.maximum(e_c - 1, 0)], 0)
    r_s = bfirst[e_c] + (s_ids - prev)
    last = jnp.maximum(total - 1, 0)
    e_last = jnp.minimum(
        jnp.sum((co <= last).astype(jnp.int32)), TOTAL_EXPERTS - 1)
    prev_last = jnp.where(e_last > 0, co[jnp.maximum(e_last - 1, 0)], 0)
    r_last = bfirst[e_last] + (last - prev_last)
    step_e = jnp.where(valid, e_c, e_last)
    step_r = jnp.where(valid, r_s, r_last)
    step_lo = jnp.where(valid, starts[e_c], 0)
    step_hi = jnp.where(valid, ends[e_c], 0)
    present = counts > 0
    slot_e = ((jnp.cumsum(present.astype(jnp.int32)) - 1) & 1)
    cand = jnp.where(present, eids, TOTAL_EXPERTS)
    sufmin = lax.cummin(cand[::-1])[::-1]
    nxt_of_e = jnp.concatenate(
        [sufmin[1:], jnp.full((1,), TOTAL_EXPERTS, jnp.int32)])
    new_e = jnp.concatenate([
        jnp.ones((1,), jnp.int32),
        (step_e[1:] != step_e[:-1]).astype(jnp.int32)])
    slot = slot_e[step_e].astype(jnp.int32)
    has_nxt = (nxt_of_e[step_e] < TOTAL_EXPERTS).astype(jnp.int32)
    nxt_e = jnp.minimum(nxt_of_e[step_e], TOTAL_EXPERTS - 1)
    return (idx, rank, step_e, step_r, step_lo, step_hi,
            slot, new_e, has_nxt, nxt_e)


def _moe_tc_body(se_ref, sr_ref, lo_ref, hi_ref, sl_ref, ne_ref, hn_ref,
                 nx_ref, xh_ref, sel_ref, bemb_ref, wg_hbm, wu_hbm, wd_hbm,
                 out_ref, wg_v, wu_v, wd_v, sg0, sg1, su0, su1, sd0, sd1):
    s = pl.program_id(0)
    lo = lo_ref[s]
    hi = hi_ref[s]
    r = sr_ref[s]
    sl = sl_ref[s]
    ne = ne_ref[s]
    sg = (sg0, sg1)
    su = (su0, su1)
    sd = (sd0, sd1)

    def fetch(e, k):
        pltpu.make_async_copy(wg_hbm.at[e], wg_v.at[k], sg[k]).start()
        pltpu.make_async_copy(wu_hbm.at[e], wu_v.at[k], su[k]).start()
        pltpu.make_async_copy(wd_hbm.at[e], wd_v.at[k], sd[k]).start()

    def wait_slot(e, k):
        pltpu.make_async_copy(wg_hbm.at[e], wg_v.at[k], sg[k]).wait()
        pltpu.make_async_copy(wu_hbm.at[e], wu_v.at[k], su[k]).wait()
        pltpu.make_async_copy(wd_hbm.at[e], wd_v.at[k], sd[k]).wait()

    @pl.when(s == 0)
    def _():
        fetch(se_ref[0], 0)

    @pl.when((ne == 1) & (hn_ref[s] == 1))
    def _():
        nx = nx_ref[s]

        @pl.when(sl == 0)
        def _():
            fetch(nx, 1)

        @pl.when(sl == 1)
        def _():
            fetch(nx, 0)

    @pl.when((ne == 1) & (sl == 0))
    def _():
        wait_slot(se_ref[s], 0)

    @pl.when((ne == 1) & (sl == 1))
    def _():
        wait_slot(se_ref[s], 1)

    def compute(k):
        bf = jnp.bfloat16
        xh = xh_ref[...].astype(bf)
        sel = sel_ref[...]
        bemb = bemb_ref[...].astype(bf)
        wgh = wg_v[k, :HIDDEN, :].astype(bf)
        wgb = wg_v[k, HIDDEN:, :].astype(bf)
        wuh = wu_v[k, :HIDDEN, :].astype(bf)
        wub = wu_v[k, HIDDEN:, :].astype(bf)
        pbg = jnp.dot(bemb, wgb, preferred_element_type=jnp.float32)
        pbu = jnp.dot(bemb, wub, preferred_element_type=jnp.float32)
        g = (jnp.dot(xh, wgh, preferred_element_type=jnp.float32)
             + pbg[0:1, :] + sel * (pbg[1:2, :] - pbg[0:1, :]))
        u = (jnp.dot(xh, wuh, preferred_element_type=jnp.float32)
             + pbu[0:1, :] + sel * (pbu[1:2, :] - pbu[0:1, :]))
        h = (g * jax.nn.sigmoid(g) * u).astype(bf)
        y = jnp.dot(h, wd_v[k].astype(bf), preferred_element_type=jnp.float32)
        gid = r * BM + lax.broadcasted_iota(jnp.int32, (BM, 1), 0)
        m = (gid >= lo) & (gid < hi)
        out_ref[...] = jnp.where(m, y, out_ref[...])

    @pl.when((hi > lo) & (sl == 0))
    def _():
        compute(0)

    @pl.when((hi > lo) & (sl == 1))
    def _():
        compute(1)


def _tc_moe(step_e, step_r, step_lo, step_hi, slot, new_e, has_nxt, nxt_e,
            xh_s, sel_col, behavior_emb, Wg, Wu, Wd):
    nmap = lambda s, *_: (0, 0)
    rmap = lambda s, se, sr, *_: (sr[s], 0)
    grid_spec = pltpu.PrefetchScalarGridSpec(
        num_scalar_prefetch=8,
        grid=(NSTEPS,),
        in_specs=[
            pl.BlockSpec((BM, HIDDEN), rmap),
            pl.BlockSpec((BM, 1), rmap),
            pl.BlockSpec((2, BEH_DIM), nmap),
            pl.BlockSpec(memory_space=pl.ANY),
            pl.BlockSpec(memory_space=pl.ANY),
            pl.BlockSpec(memory_space=pl.ANY),
        ],
        out_specs=pl.BlockSpec((BM, HIDDEN), rmap),
        scratch_shapes=[
            pltpu.VMEM((2, HIDDEN + BEH_DIM, INTER), jnp.float32),
            pltpu.VMEM((2, HIDDEN + BEH_DIM, INTER), jnp.float32),
            pltpu.VMEM((2, INTER, HIDDEN), jnp.float32),
            pltpu.SemaphoreType.DMA, pltpu.SemaphoreType.DMA,
            pltpu.SemaphoreType.DMA, pltpu.SemaphoreType.DMA,
            pltpu.SemaphoreType.DMA, pltpu.SemaphoreType.DMA,
        ],
    )
    return pl.pallas_call(
        _moe_tc_body,
        grid_spec=grid_spec,
        out_shape=jax.ShapeDtypeStruct((T, HIDDEN), jnp.float32),
        compiler_params=pltpu.CompilerParams(
            dimension_semantics=("arbitrary",)),
    )(step_e, step_r, step_lo, step_hi, slot, new_e, has_nxt, nxt_e,
      xh_s, sel_col, behavior_emb, Wg, Wu, Wd)


def _sc_dispatch(hidden_states, rank):
    mesh = plsc.VectorSubcoreMesh(core_axis_name="c", subcore_axis_name="s")

    @functools.partial(
        pl.kernel, mesh=mesh,
        out_type=jax.ShapeDtypeStruct((T, HIDDEN), jnp.float32),
        scratch_types=[pltpu.VMEM((CH,), jnp.int32),
                       pltpu.VMEM((CH,), jnp.int32),
                       pltpu.VMEM((CH, HIDDEN), jnp.float32),
                       pltpu.VMEM((CH, HIDDEN), jnp.float32),
                       pltpu.SemaphoreType.DMA, pltpu.SemaphoreType.DMA,
                       pltpu.SemaphoreType.DMA, pltpu.SemaphoreType.DMA,
                       pltpu.SemaphoreType.DMA, pltpu.SemaphoreType.DMA],
    )
    def dispatch_k(hid_hbm, rank_hbm, xh_hbm, r0, r1, h0, h1,
                   sr0, sr1, sh0, sh1, w0, w1):
        wid = lax.axis_index("s") * 2 + lax.axis_index("c")
        base = wid * ROWS_PER_W
        rb = (r0, r1)
        hb = (h0, h1)
        sr = (sr0, sr1)
        sh = (sh0, sh1)
        ws = (w0, w1)

        def start(c):
            buf = c & 1
            return (pltpu.async_copy(rank_hbm.at[pl.ds(base + c * CH, CH)],
                                     rb[buf], sr[buf]),
                    pltpu.async_copy(hid_hbm.at[pl.ds(base + c * CH, CH)],
                                     hb[buf], sh[buf]))

        pend = start(0)
        w_pend = [None, None]
        for c in range(NCH):
            buf = c & 1
            for p in pend:
                p.wait()
            if c + 1 < NCH:
                nbuf = (c + 1) & 1
                if w_pend[nbuf] is not None:
                    w_pend[nbuf].wait()
                    w_pend[nbuf] = None
                pend = start(c + 1)
            w_pend[buf] = pltpu.async_copy(hb[buf], xh_hbm.at[rb[buf]],
                                           ws[buf])
        for p in w_pend:
            if p is not None:
                p.wait()

    return dispatch_k(hidden_states, rank)


def _sc_scatter(y_sorted, perm):
    mesh = plsc.VectorSubcoreMesh(core_axis_name="c", subcore_axis_name="s")

    @functools.partial(
        pl.kernel, mesh=mesh,
        out_type=jax.ShapeDtypeStruct((T, HIDDEN), jnp.float32),
        scratch_types=[pltpu.VMEM((CH,), jnp.int32),
                       pltpu.VMEM((CH,), jnp.int32),
                       pltpu.VMEM((CH, HIDDEN), jnp.float32),
                       pltpu.VMEM((CH, HIDDEN), jnp.float32),
                       pltpu.SemaphoreType.DMA, pltpu.SemaphoreType.DMA,
                       pltpu.SemaphoreType.DMA, pltpu.SemaphoreType.DMA,
                       pltpu.SemaphoreType.DMA, pltpu.SemaphoreType.DMA],
    )
    def scatter_k(y_hbm, perm_hbm, out_hbm, i0, i1, y0, y1,
                  ri0, ri1, ry0, ry1, w0, w1):
        wid = lax.axis_index("s") * 2 + lax.axis_index("c")
        base = wid * ROWS_PER_W
        ib = (i0, i1)
        yb = (y0, y1)
        ri = (ri0, ri1)
        ry = (ry0, ry1)
        ws = (w0, w1)

        def start(c):
            buf = c & 1
            return (pltpu.async_copy(perm_hbm.at[pl.ds(base + c * CH, CH)],
                                     ib[buf], ri[buf]),
                    pltpu.async_copy(y_hbm.at[pl.ds(base + c * CH, CH)],
                                     yb[buf], ry[buf]))

        pend = start(0)
        w_pend = [None, None]
        for c in range(NCH):
            buf = c & 1
            pend[0].wait()
            pend[1].wait()
            if c + 1 < NCH:
                nbuf = (c + 1) & 1
                if w_pend[nbuf] is not None:
                    w_pend[nbuf].wait()
                    w_pend[nbuf] = None
                pend = start(c + 1)
            w_pend[buf] = pltpu.async_copy(yb[buf], out_hbm.at[ib[buf]],
                                           ws[buf])
        for p in w_pend:
            if p is not None:
                p.wait()

    return scatter_k(y_sorted, perm)


def kernel(hidden_states, position_index, behavior_index, action_index,
           behavior_emb, Wg, Wu, Wd):
    (_, rank, step_e, step_r, step_lo, step_hi,
     slot, new_e, has_nxt, nxt_e) = _route_meta(action_index, position_index)
    perm = jnp.zeros((T,), jnp.int32).at[rank].set(
        jnp.arange(T, dtype=jnp.int32))
    sel_col = jnp.zeros((T,), jnp.float32).at[rank].set(
        behavior_index.astype(jnp.float32)).reshape(T, 1)
    xh_s = _sc_dispatch(hidden_states, rank)
    y_s = _tc_moe(step_e, step_r, step_lo, step_hi, slot, new_e, has_nxt,
                  nxt_e, xh_s, sel_col, behavior_emb, Wg, Wu, Wd)
    return _sc_scatter(y_s, perm)


# speedup vs baseline: 1.0434x; 1.0434x over previous
import functools

import jax
import jax.numpy as jnp
from jax import lax
from jax.experimental import pallas as pl
from jax.experimental.pallas import tpu as pltpu
from jax.experimental.pallas import tpu_sc as plsc

NUM_EXPERTS = 8
TOTAL_EXPERTS = 8
HIDDEN = 2048
BEH_DIM = 64
INTER = 768
T = 2048

BM = 128
NBLK = T // BM
NSTEPS = NBLK + TOTAL_EXPERTS - 1

BEH_PAD = 128

NW = 32
ROWS_PER_W = T // NW
CH = 16
NCH = ROWS_PER_W // CH


def _route_meta(action_index, position_index):
    idx = jnp.maximum(
        (NUM_EXPERTS - 1) * (action_index.astype(jnp.int32) - 1)
        + position_index.astype(jnp.int32), 0)
    eids = jnp.arange(TOTAL_EXPERTS, dtype=jnp.int32)
    oh = (idx[None, :] == eids[:, None]).astype(jnp.int32)
    csum = jnp.cumsum(oh, axis=1)
    counts = csum[:, -1]
    ends = jnp.cumsum(counts)
    starts = ends - counts
    cnt_before = jnp.take_along_axis(csum, idx[None, :], axis=0)[0] - 1
    rank = starts[idx] + cnt_before
    bfirst = starts // BM
    bcnt = jnp.where(counts > 0, (ends + BM - 1) // BM - bfirst, 0)
    co = jnp.cumsum(bcnt)
    s_ids = jnp.arange(NSTEPS, dtype=jnp.int32)
    e_s = jnp.sum((s_ids[:, None] >= co[None, :]).astype(jnp.int32), axis=1)
    total = co[TOTAL_EXPERTS - 1]
    valid = s_ids < total
    e_c = jnp.minimum(e_s, TOTAL_EXPERTS - 1)
    prev = jnp.where(e_c > 0, co[jnp.maximum(e_c - 1, 0)], 0)
    r_s = bfirst[e_c] + (s_ids - prev)
    last = jnp.maximum(total - 1, 0)
    e_last = jnp.minimum(
        jnp.sum((co <= last).astype(jnp.int32)), TOTAL_EXPERTS - 1)
    prev_last = jnp.where(e_last > 0, co[jnp.maximum(e_last - 1, 0)], 0)
    r_last = bfirst[e_last] + (last - prev_last)
    step_e = jnp.where(valid, e_c, e_last)
    step_r = jnp.where(valid, r_s, r_last)
    step_lo = jnp.where(valid, starts[e_c], 0)
    step_hi = jnp.where(valid, ends[e_c], 0)
    present = counts > 0
    slot_e = ((jnp.cumsum(present.astype(jnp.int32)) - 1) & 1)
    cand = jnp.where(present, eids, TOTAL_EXPERTS)
    sufmin = lax.cummin(cand[::-1])[::-1]
    nxt_of_e = jnp.concatenate(
        [sufmin[1:], jnp.full((1,), TOTAL_EXPERTS, jnp.int32)])
    new_e = jnp.concatenate([
        jnp.ones((1,), jnp.int32),
        (step_e[1:] != step_e[:-1]).astype(jnp.int32)])
    slot = slot_e[step_e].astype(jnp.int32)
    has_nxt = (nxt_of_e[step_e] < TOTAL_EXPERTS).astype(jnp.int32)
    nxt_e = jnp.minimum(nxt_of_e[step_e], TOTAL_EXPERTS - 1)
    return (idx, rank, step_e, step_r, step_lo, step_hi,
            slot, new_e, has_nxt, nxt_e)


def _moe_tc_body(se_ref, sr_ref, lo_ref, hi_ref, sl_ref, ne_ref, hn_ref,
                 nx_ref, xh_ref, sel_ref, bemb_ref, wg_hbm, wu_hbm, wd_hbm,
                 out_ref, wg_v, wu_v, wd_v, sg0, sg1, su0, su1, sd0, sd1):
    s = pl.program_id(0)
    lo = lo_ref[s]
    hi = hi_ref[s]
    r = sr_ref[s]
    sl = sl_ref[s]
    ne = ne_ref[s]
    sg = (sg0, sg1)
    su = (su0, su1)
    sd = (sd0, sd1)

    def fetch(e, k):
        pltpu.make_async_copy(wg_hbm.at[e], wg_v.at[k], sg[k]).start()
        pltpu.make_async_copy(wu_hbm.at[e], wu_v.at[k], su[k]).start()
        pltpu.make_async_copy(wd_hbm.at[e], wd_v.at[k], sd[k]).start()

    def wait_slot(e, k):
        pltpu.make_async_copy(wg_hbm.at[e], wg_v.at[k], sg[k]).wait()
        pltpu.make_async_copy(wu_hbm.at[e], wu_v.at[k], su[k]).wait()
        pltpu.make_async_copy(wd_hbm.at[e], wd_v.at[k], sd[k]).wait()

    @pl.when(s == 0)
    def _():
        fetch(se_ref[0], 0)

    @pl.when((ne == 1) & (hn_ref[s] == 1))
    def _():
        nx = nx_ref[s]

        @pl.when(sl == 0)
        def _():
            fetch(nx, 1)

        @pl.when(sl == 1)
        def _():
            fetch(nx, 0)

    @pl.when((ne == 1) & (sl == 0))
    def _():
        wait_slot(se_ref[s], 0)

    @pl.when((ne == 1) & (sl == 1))
    def _():
        wait_slot(se_ref[s], 1)

    def compute(k):
        bf = jnp.bfloat16
        xh = xh_ref[...].astype(bf)
        sel = sel_ref[...]
        bemb = bemb_ref[...].astype(bf)
        wgh = wg_v[k, :HIDDEN, :].astype(bf)
        wgb = wg_v[k, HIDDEN:, :].astype(bf)
        wuh = wu_v[k, :HIDDEN, :].astype(bf)
        wub = wu_v[k, HIDDEN:, :].astype(bf)
        pbg = jnp.dot(bemb, wgb, preferred_element_type=jnp.float32)
        pbu = jnp.dot(bemb, wub, preferred_element_type=jnp.float32)
        g = (jnp.dot(xh, wgh, preferred_element_type=jnp.float32)
             + pbg[0:1, :] + sel * (pbg[1:2, :] - pbg[0:1, :]))
        u = (jnp.dot(xh, wuh, preferred_element_type=jnp.float32)
             + pbu[0:1, :] + sel * (pbu[1:2, :] - pbu[0:1, :]))
        h = (g * jax.nn.sigmoid(g) * u).astype(bf)
        y = jnp.dot(h, wd_v[k].astype(bf), preferred_element_type=jnp.float32)
        gid = r * BM + lax.broadcasted_iota(jnp.int32, (BM, 1), 0)
        m = (gid >= lo) & (gid < hi)
        out_ref[...] = jnp.where(m, y, out_ref[...])

    @pl.when((hi > lo) & (sl == 0))
    def _():
        compute(0)

    @pl.when((hi > lo) & (sl == 1))
    def _():
        compute(1)


def _tc_moe(step_e, step_r, step_lo, step_hi, slot, new_e, has_nxt, nxt_e,
            xh_s, sel_col, behavior_emb, Wg, Wu, Wd):
    nmap = lambda s, *_: (0, 0)
    rmap = lambda s, se, sr, *_: (sr[s], 0)
    grid_spec = pltpu.PrefetchScalarGridSpec(
        num_scalar_prefetch=8,
        grid=(NSTEPS,),
        in_specs=[
            pl.BlockSpec((BM, HIDDEN), rmap),
            pl.BlockSpec((BM, 1), rmap),
            pl.BlockSpec((2, BEH_DIM), nmap),
            pl.BlockSpec(memory_space=pl.ANY),
            pl.BlockSpec(memory_space=pl.ANY),
            pl.BlockSpec(memory_space=pl.ANY),
        ],
        out_specs=pl.BlockSpec((BM, HIDDEN), rmap),
        scratch_shapes=[
            pltpu.VMEM((2, HIDDEN + BEH_DIM, INTER), jnp.float32),
            pltpu.VMEM((2, HIDDEN + BEH_DIM, INTER), jnp.float32),
            pltpu.VMEM((2, INTER, HIDDEN), jnp.float32),
            pltpu.SemaphoreType.DMA, pltpu.SemaphoreType.DMA,
            pltpu.SemaphoreType.DMA, pltpu.SemaphoreType.DMA,
            pltpu.SemaphoreType.DMA, pltpu.SemaphoreType.DMA,
        ],
    )
    return pl.pallas_call(
        _moe_tc_body,
        grid_spec=grid_spec,
        out_shape=jax.ShapeDtypeStruct((T, HIDDEN), jnp.float32),
        compiler_params=pltpu.CompilerParams(
            dimension_semantics=("arbitrary",)),
    )(step_e, step_r, step_lo, step_hi, slot, new_e, has_nxt, nxt_e,
      xh_s, sel_col, behavior_emb, Wg, Wu, Wd)


def _sc_dispatch(hidden_states, rank):
    mesh = plsc.VectorSubcoreMesh(core_axis_name="c", subcore_axis_name="s")

    @functools.partial(
        pl.kernel, mesh=mesh,
        out_type=jax.ShapeDtypeStruct((T, HIDDEN), jnp.float32),
        scratch_types=[pltpu.VMEM((CH,), jnp.int32),
                       pltpu.VMEM((CH,), jnp.int32),
                       pltpu.VMEM((CH, HIDDEN), jnp.float32),
                       pltpu.VMEM((CH, HIDDEN), jnp.float32),
                       pltpu.SemaphoreType.DMA, pltpu.SemaphoreType.DMA,
                       pltpu.SemaphoreType.DMA, pltpu.SemaphoreType.DMA,
                       pltpu.SemaphoreType.DMA, pltpu.SemaphoreType.DMA],
    )
    def dispatch_k(hid_hbm, rank_hbm, xh_hbm, r0, r1, h0, h1,
                   sr0, sr1, sh0, sh1, w0, w1):
        wid = lax.axis_index("s") * 2 + lax.axis_index("c")
        base = wid * ROWS_PER_W
        rb = (r0, r1)
        hb = (h0, h1)
        sr = (sr0, sr1)
        sh = (sh0, sh1)
        ws = (w0, w1)

        def start(c):
            buf = c & 1
            return (pltpu.async_copy(rank_hbm.at[pl.ds(base + c * CH, CH)],
                                     rb[buf], sr[buf]),
                    pltpu.async_copy(hid_hbm.at[pl.ds(base + c * CH, CH)],
                                     hb[buf], sh[buf]))

        pend = start(0)
        w_pend = [None, None]
        for c in range(NCH):
            buf = c & 1
            for p in pend:
                p.wait()
            if c + 1 < NCH:
                nbuf = (c + 1) & 1
                if w_pend[nbuf] is not None:
                    w_pend[nbuf].wait()
                    w_pend[nbuf] = None
                pend = start(c + 1)
            w_pend[buf] = pltpu.async_copy(hb[buf], xh_hbm.at[rb[buf]],
                                           ws[buf])
        for p in w_pend:
            if p is not None:
                p.wait()

    return dispatch_k(hidden_states, rank)


def _sc_scatter(y_sorted, perm):
    mesh = plsc.VectorSubcoreMesh(core_axis_name="c", subcore_axis_name="s")

    @functools.partial(
        pl.kernel, mesh=mesh,
        out_type=jax.ShapeDtypeStruct((T, HIDDEN), jnp.float32),
        scratch_types=[pltpu.VMEM((CH,), jnp.int32),
                       pltpu.VMEM((CH,), jnp.int32),
                       pltpu.VMEM((CH, HIDDEN), jnp.float32),
                       pltpu.VMEM((CH, HIDDEN), jnp.float32),
                       pltpu.SemaphoreType.DMA, pltpu.SemaphoreType.DMA,
                       pltpu.SemaphoreType.DMA, pltpu.SemaphoreType.DMA,
                       pltpu.SemaphoreType.DMA, pltpu.SemaphoreType.DMA],
    )
    def scatter_k(y_hbm, perm_hbm, out_hbm, i0, i1, y0, y1,
                  ri0, ri1, ry0, ry1, w0, w1):
        wid = lax.axis_index("s") * 2 + lax.axis_index("c")
        base = wid * ROWS_PER_W
        ib = (i0, i1)
        yb = (y0, y1)
        ri = (ri0, ri1)
        ry = (ry0, ry1)
        ws = (w0, w1)

        def start(c):
            buf = c & 1
            return (pltpu.async_copy(perm_hbm.at[pl.ds(base + c * CH, CH)],
                                     ib[buf], ri[buf]),
                    pltpu.async_copy(y_hbm.at[pl.ds(base + c * CH, CH)],
                                     yb[buf], ry[buf]))

        pend = start(0)
        w_pend = [None, None]
        for c in range(NCH):
            buf = c & 1
            pend[0].wait()
            pend[1].wait()
            if c + 1 < NCH:
                nbuf = (c + 1) & 1
                if w_pend[nbuf] is not None:
                    w_pend[nbuf].wait()
                    w_pend[nbuf] = None
                pend = start(c + 1)
            w_pend[buf] = pltpu.async_copy(yb[buf], out_hbm.at[ib[buf]],
                                           ws[buf])
        for p in w_pend:
            if p is not None:
                p.wait()

    return scatter_k(y_sorted, perm)


def kernel(hidden_states, position_index, behavior_index, action_index,
           behavior_emb, Wg, Wu, Wd):
    (idx, rank, step_e, step_r, step_lo, step_hi,
     slot, new_e, has_nxt, nxt_e) = _route_meta(action_index, position_index)
    perm = jnp.argsort(idx, stable=True).astype(jnp.int32)
    sel_col = behavior_index.astype(jnp.float32)[perm].reshape(T, 1)
    xh_s = _sc_dispatch(hidden_states, rank)
    y_s = _tc_moe(step_e, step_r, step_lo, step_hi, slot, new_e, has_nxt,
                  nxt_e, xh_s, sel_col, behavior_emb, Wg, Wu, Wd)
    return _sc_scatter(y_s, perm)


# speedup vs baseline: 1.1013x; 1.0556x over previous
import functools

import jax
import jax.numpy as jnp
from jax import lax
from jax.experimental import pallas as pl
from jax.experimental.pallas import tpu as pltpu
from jax.experimental.pallas import tpu_sc as plsc

NUM_EXPERTS = 8
TOTAL_EXPERTS = 8
HIDDEN = 2048
BEH_DIM = 64
INTER = 768
T = 2048

BM = 128
NBLK = T // BM
NSTEPS = NBLK + TOTAL_EXPERTS - 1

BEH_PAD = 128

NW = 32
ROWS_PER_W = T // NW
CH = 16
NCH = ROWS_PER_W // CH


def _route_meta(action_index, position_index):
    idx = jnp.maximum(
        (NUM_EXPERTS - 1) * (action_index.astype(jnp.int32) - 1)
        + position_index.astype(jnp.int32), 0)
    eids = jnp.arange(TOTAL_EXPERTS, dtype=jnp.int32)
    oh = (idx[None, :] == eids[:, None]).astype(jnp.int32)
    csum = jnp.cumsum(oh, axis=1)
    counts = csum[:, -1]
    ends = jnp.cumsum(counts)
    starts = ends - counts
    rank = jnp.sum(oh * (csum + starts[:, None]), axis=0) - 1
    bfirst = starts // BM
    bcnt = jnp.where(counts > 0, (ends + BM - 1) // BM - bfirst, 0)
    co = jnp.cumsum(bcnt)
    s_ids = jnp.arange(NSTEPS, dtype=jnp.int32)
    e_s = jnp.sum((s_ids[:, None] >= co[None, :]).astype(jnp.int32), axis=1)
    total = co[TOTAL_EXPERTS - 1]
    valid = s_ids < total
    e_c = jnp.minimum(e_s, TOTAL_EXPERTS - 1)
    prev = jnp.where(e_c > 0, co[jnp.maximum(e_c - 1, 0)], 0)
    r_s = bfirst[e_c] + (s_ids - prev)
    last = jnp.maximum(total - 1, 0)
    e_last = jnp.minimum(
        jnp.sum((co <= last).astype(jnp.int32)), TOTAL_EXPERTS - 1)
    prev_last = jnp.where(e_last > 0, co[jnp.maximum(e_last - 1, 0)], 0)
    r_last = bfirst[e_last] + (last - prev_last)
    step_e = jnp.where(valid, e_c, e_last)
    step_r = jnp.where(valid, r_s, r_last)
    step_lo = jnp.where(valid, starts[e_c], 0)
    step_hi = jnp.where(valid, ends[e_c], 0)
    present = counts > 0
    slot_e = ((jnp.cumsum(present.astype(jnp.int32)) - 1) & 1)
    cand = jnp.where(present, eids, TOTAL_EXPERTS)
    sufmin = lax.cummin(cand[::-1])[::-1]
    nxt_of_e = jnp.concatenate(
        [sufmin[1:], jnp.full((1,), TOTAL_EXPERTS, jnp.int32)])
    new_e = jnp.concatenate([
        jnp.ones((1,), jnp.int32),
        (step_e[1:] != step_e[:-1]).astype(jnp.int32)])
    slot = slot_e[step_e].astype(jnp.int32)
    has_nxt = (nxt_of_e[step_e] < TOTAL_EXPERTS).astype(jnp.int32)
    nxt_e = jnp.minimum(nxt_of_e[step_e], TOTAL_EXPERTS - 1)
    return (idx, rank, step_e, step_r, step_lo, step_hi,
            slot, new_e, has_nxt, nxt_e)


def _moe_tc_body(se_ref, sr_ref, lo_ref, hi_ref, sl_ref, ne_ref, hn_ref,
                 nx_ref, xh_ref, sel_ref, bemb_ref, wg_hbm, wu_hbm, wd_hbm,
                 out_ref, wg_v, wu_v, wd_v, sg0, sg1, su0, su1, sd0, sd1):
    s = pl.program_id(0)
    lo = lo_ref[s]
    hi = hi_ref[s]
    r = sr_ref[s]
    sl = sl_ref[s]
    ne = ne_ref[s]
    sg = (sg0, sg1)
    su = (su0, su1)
    sd = (sd0, sd1)

    def fetch(e, k):
        pltpu.make_async_copy(wg_hbm.at[e], wg_v.at[k], sg[k]).start()
        pltpu.make_async_copy(wu_hbm.at[e], wu_v.at[k], su[k]).start()
        pltpu.make_async_copy(wd_hbm.at[e], wd_v.at[k], sd[k]).start()

    def wait_slot(e, k):
        pltpu.make_async_copy(wg_hbm.at[e], wg_v.at[k], sg[k]).wait()
        pltpu.make_async_copy(wu_hbm.at[e], wu_v.at[k], su[k]).wait()
        pltpu.make_async_copy(wd_hbm.at[e], wd_v.at[k], sd[k]).wait()

    @pl.when(s == 0)
    def _():
        fetch(se_ref[0], 0)

    @pl.when((ne == 1) & (hn_ref[s] == 1))
    def _():
        nx = nx_ref[s]

        @pl.when(sl == 0)
        def _():
            fetch(nx, 1)

        @pl.when(sl == 1)
        def _():
            fetch(nx, 0)

    @pl.when((ne == 1) & (sl == 0))
    def _():
        wait_slot(se_ref[s], 0)

    @pl.when((ne == 1) & (sl == 1))
    def _():
        wait_slot(se_ref[s], 1)

    def compute(k):
        bf = jnp.bfloat16
        xh = xh_ref[...].astype(bf)
        sel = sel_ref[...]
        bemb = bemb_ref[...].astype(bf)
        wgh = wg_v[k, :HIDDEN, :].astype(bf)
        wgb = wg_v[k, HIDDEN:, :].astype(bf)
        wuh = wu_v[k, :HIDDEN, :].astype(bf)
        wub = wu_v[k, HIDDEN:, :].astype(bf)
        pbg = jnp.dot(bemb, wgb, preferred_element_type=jnp.float32)
        pbu = jnp.dot(bemb, wub, preferred_element_type=jnp.float32)
        g = (jnp.dot(xh, wgh, preferred_element_type=jnp.float32)
             + pbg[0:1, :] + sel * (pbg[1:2, :] - pbg[0:1, :]))
        u = (jnp.dot(xh, wuh, preferred_element_type=jnp.float32)
             + pbu[0:1, :] + sel * (pbu[1:2, :] - pbu[0:1, :]))
        h = (g * jax.nn.sigmoid(g) * u).astype(bf)
        y = jnp.dot(h, wd_v[k].astype(bf), preferred_element_type=jnp.float32)
        gid = r * BM + lax.broadcasted_iota(jnp.int32, (BM, 1), 0)
        m = (gid >= lo) & (gid < hi)
        out_ref[...] = jnp.where(m, y, out_ref[...])

    @pl.when((hi > lo) & (sl == 0))
    def _():
        compute(0)

    @pl.when((hi > lo) & (sl == 1))
    def _():
        compute(1)


def _tc_moe(step_e, step_r, step_lo, step_hi, slot, new_e, has_nxt, nxt_e,
            xh_s, sel_col, behavior_emb, Wg, Wu, Wd):
    nmap = lambda s, *_: (0, 0)
    rmap = lambda s, se, sr, *_: (sr[s], 0)
    grid_spec = pltpu.PrefetchScalarGridSpec(
        num_scalar_prefetch=8,
        grid=(NSTEPS,),
        in_specs=[
            pl.BlockSpec((BM, HIDDEN), rmap),
            pl.BlockSpec((BM, 1), rmap),
            pl.BlockSpec((2, BEH_DIM), nmap),
            pl.BlockSpec(memory_space=pl.ANY),
            pl.BlockSpec(memory_space=pl.ANY),
            pl.BlockSpec(memory_space=pl.ANY),
        ],
        out_specs=pl.BlockSpec((BM, HIDDEN), rmap),
        scratch_shapes=[
            pltpu.VMEM((2, HIDDEN + BEH_DIM, INTER), jnp.float32),
            pltpu.VMEM((2, HIDDEN + BEH_DIM, INTER), jnp.float32),
            pltpu.VMEM((2, INTER, HIDDEN), jnp.float32),
            pltpu.SemaphoreType.DMA, pltpu.SemaphoreType.DMA,
            pltpu.SemaphoreType.DMA, pltpu.SemaphoreType.DMA,
            pltpu.SemaphoreType.DMA, pltpu.SemaphoreType.DMA,
        ],
    )
    return pl.pallas_call(
        _moe_tc_body,
        grid_spec=grid_spec,
        out_shape=jax.ShapeDtypeStruct((T, HIDDEN), jnp.float32),
        compiler_params=pltpu.CompilerParams(
            dimension_semantics=("arbitrary",)),
    )(step_e, step_r, step_lo, step_hi, slot, new_e, has_nxt, nxt_e,
      xh_s, sel_col, behavior_emb, Wg, Wu, Wd)


def _sc_dispatch(hidden_states, rank):
    mesh = plsc.VectorSubcoreMesh(core_axis_name="c", subcore_axis_name="s")

    @functools.partial(
        pl.kernel, mesh=mesh,
        out_type=jax.ShapeDtypeStruct((T, HIDDEN), jnp.float32),
        scratch_types=[pltpu.VMEM((CH,), jnp.int32),
                       pltpu.VMEM((CH,), jnp.int32),
                       pltpu.VMEM((CH, HIDDEN), jnp.float32),
                       pltpu.VMEM((CH, HIDDEN), jnp.float32),
                       pltpu.SemaphoreType.DMA, pltpu.SemaphoreType.DMA,
                       pltpu.SemaphoreType.DMA, pltpu.SemaphoreType.DMA,
                       pltpu.SemaphoreType.DMA, pltpu.SemaphoreType.DMA],
    )
    def dispatch_k(hid_hbm, rank_hbm, xh_hbm, r0, r1, h0, h1,
                   sr0, sr1, sh0, sh1, w0, w1):
        wid = lax.axis_index("s") * 2 + lax.axis_index("c")
        base = wid * ROWS_PER_W
        rb = (r0, r1)
        hb = (h0, h1)
        sr = (sr0, sr1)
        sh = (sh0, sh1)
        ws = (w0, w1)

        def start(c):
            buf = c & 1
            return (pltpu.async_copy(rank_hbm.at[pl.ds(base + c * CH, CH)],
                                     rb[buf], sr[buf]),
                    pltpu.async_copy(hid_hbm.at[pl.ds(base + c * CH, CH)],
                                     hb[buf], sh[buf]))

        pend = start(0)
        w_pend = [None, None]
        for c in range(NCH):
            buf = c & 1
            for p in pend:
                p.wait()
            if c + 1 < NCH:
                nbuf = (c + 1) & 1
                if w_pend[nbuf] is not None:
                    w_pend[nbuf].wait()
                    w_pend[nbuf] = None
                pend = start(c + 1)
            w_pend[buf] = pltpu.async_copy(hb[buf], xh_hbm.at[rb[buf]],
                                           ws[buf])
        for p in w_pend:
            if p is not None:
                p.wait()

    return dispatch_k(hidden_states, rank)


def _sc_scatter(y_sorted, perm):
    mesh = plsc.VectorSubcoreMesh(core_axis_name="c", subcore_axis_name="s")

    @functools.partial(
        pl.kernel, mesh=mesh,
        out_type=jax.ShapeDtypeStruct((T, HIDDEN), jnp.float32),
        scratch_types=[pltpu.VMEM((CH,), jnp.int32),
                       pltpu.VMEM((CH,), jnp.int32),
                       pltpu.VMEM((CH, HIDDEN), jnp.float32),
                       pltpu.VMEM((CH, HIDDEN), jnp.float32),
                       pltpu.SemaphoreType.DMA, pltpu.SemaphoreType.DMA,
                       pltpu.SemaphoreType.DMA, pltpu.SemaphoreType.DMA,
                       pltpu.SemaphoreType.DMA, pltpu.SemaphoreType.DMA],
    )
    def scatter_k(y_hbm, perm_hbm, out_hbm, i0, i1, y0, y1,
                  ri0, ri1, ry0, ry1, w0, w1):
        wid = lax.axis_index("s") * 2 + lax.axis_index("c")
        base = wid * ROWS_PER_W
        ib = (i0, i1)
        yb = (y0, y1)
        ri = (ri0, ri1)
        ry = (ry0, ry1)
        ws = (w0, w1)

        def start(c):
            buf = c & 1
            return (pltpu.async_copy(perm_hbm.at[pl.ds(base + c * CH, CH)],
                                     ib[buf], ri[buf]),
                    pltpu.async_copy(y_hbm.at[pl.ds(base + c * CH, CH)],
                                     yb[buf], ry[buf]))

        pend = start(0)
        w_pend = [None, None]
        for c in range(NCH):
            buf = c & 1
            pend[0].wait()
            pend[1].wait()
            if c + 1 < NCH:
                nbuf = (c + 1) & 1
                if w_pend[nbuf] is not None:
                    w_pend[nbuf].wait()
                    w_pend[nbuf] = None
                pend = start(c + 1)
            w_pend[buf] = pltpu.async_copy(yb[buf], out_hbm.at[ib[buf]],
                                           ws[buf])
        for p in w_pend:
            if p is not None:
                p.wait()

    return scatter_k(y_sorted, perm)


def kernel(hidden_states, position_index, behavior_index, action_index,
           behavior_emb, Wg, Wu, Wd):
    (idx, rank, step_e, step_r, step_lo, step_hi,
     slot, new_e, has_nxt, nxt_e) = _route_meta(action_index, position_index)
    perm = jnp.argsort(idx, stable=True).astype(jnp.int32)
    sel_col = behavior_index.astype(jnp.float32)[perm].reshape(T, 1)
    xh_s = _sc_dispatch(hidden_states, rank)
    y_s = _tc_moe(step_e, step_r, step_lo, step_hi, slot, new_e, has_nxt,
                  nxt_e, xh_s, sel_col, behavior_emb, Wg, Wu, Wd)
    return _sc_scatter(y_s, perm)
